# baseline (speedup 1.0000x reference)
import functools

import jax
import jax.numpy as jnp
from jax import lax
from jax.experimental import pallas as pl
from jax.experimental.pallas import tpu as pltpu, tpu_sc as plsc

IDXW = 128


def _emb_call(B, D, n_chunks, K):
    mesh = plsc.VectorSubcoreMesh(core_axis_name="c", subcore_axis_name="s")
    info = plsc.get_sparse_core_info()
    NC, NS = info.num_cores, info.num_subcores
    rows_per_w = n_chunks * K

    @functools.partial(
        pl.kernel,
        mesh=mesh,
        out_type=jax.ShapeDtypeStruct((B, D), jnp.float32),
        compiler_params=pltpu.CompilerParams(use_tc_tiling_on_sc=False),
        scratch_types=[
            pltpu.VMEM((K, IDXW), jnp.int32),
            pltpu.VMEM((K * IDXW, D), jnp.float32),
            pltpu.SemaphoreType.DMA,
        ],
    )
    def emb(table_hbm, idx_hbm, out_hbm, idx_v, rows_v, sem):
        wid = lax.axis_index("s") * NC + lax.axis_index("c")
        row0 = wid * rows_per_w

        def body(i, carry):
            r = row0 + i * K
            pltpu.sync_copy(idx_hbm.at[pl.ds(r, K)], idx_v)
            cps = [
                pltpu.async_copy(
                    table_hbm.at[idx_v.at[j]],
                    rows_v.at[pl.ds(j * IDXW, IDXW)],
                    sem,
                )
                for j in range(K)
            ]
            for cp in cps:
                cp.wait()
            pltpu.sync_copy(rows_v, out_hbm.at[pl.ds(r * IDXW, K * IDXW)])
            return carry

        lax.fori_loop(0, n_chunks, body, 0)

    return emb


def kernel(data, table):
    B0, S = data.shape
    V, D = table.shape
    B = B0 * S
    NW = 32
    assert B % (NW * IDXW) == 0
    rows_per_w = B // (NW * IDXW)
    K = 8
    n_chunks = rows_per_w // K
    idx2d = data.reshape(B // IDXW, IDXW)
    out = _emb_call(B, D, n_chunks, K)(table, idx2d)
    return out.reshape(B0, S, D)


# speedup vs baseline: 1.0097x; 1.0097x over previous
import functools

import jax
import jax.numpy as jnp
from jax import lax
from jax.experimental import pallas as pl
from jax.experimental.pallas import tpu as pltpu, tpu_sc as plsc

IDXW = 128


def _emb_call(B, D, rows_per_w, K):
    mesh = plsc.VectorSubcoreMesh(core_axis_name="c", subcore_axis_name="s")
    info = plsc.get_sparse_core_info()
    NC = info.num_cores
    C = K * IDXW
    n_chunks = rows_per_w // K
    n_pairs = n_chunks // 2

    @functools.partial(
        pl.kernel,
        mesh=mesh,
        out_type=jax.ShapeDtypeStruct((B, D), jnp.float32),
        compiler_params=pltpu.CompilerParams(use_tc_tiling_on_sc=False),
        scratch_types=[
            pltpu.VMEM((rows_per_w, IDXW), jnp.int32),
            pltpu.VMEM((2, C, D), jnp.float32),
            pltpu.SemaphoreType.DMA,
            pltpu.SemaphoreType.DMA,
            pltpu.SemaphoreType.DMA,
            pltpu.SemaphoreType.DMA,
        ],
    )
    def emb(table_hbm, idx_hbm, out_hbm, idx_v, rows_v, g0, g1, o0, o1):
        gsem = (g0, g1)
        osem = (o0, o1)
        wid = lax.axis_index("s") * NC + lax.axis_index("c")
        row0 = wid * rows_per_w

        pltpu.sync_copy(idx_hbm.at[pl.ds(row0, rows_per_w)], idx_v)

        def fire(t, b):
            for j in range(K):
                pltpu.async_copy(
                    table_hbm.at[idx_v.at[t * K + j]],
                    rows_v.at[b, pl.ds(j * IDXW, IDXW)],
                    gsem[b],
                )

        def drain_gather(b):
            for _ in range(K):
                pltpu.make_async_copy(
                    table_hbm.at[idx_v.at[0]],
                    rows_v.at[b, pl.ds(0, IDXW)],
                    gsem[b],
                ).wait()

        def start_out(t, b):
            pltpu.async_copy(
                rows_v.at[b],
                out_hbm.at[pl.ds((row0 + t * K) * IDXW, C)],
                osem[b],
            )

        def drain_out(b):
            pltpu.make_async_copy(
                rows_v.at[b],
                out_hbm.at[pl.ds(row0 * IDXW, C)],
                osem[b],
            ).wait()

        fire(0, 0)
        fire(1, 1)

        def body(g, carry):
            t0 = 2 * g
            for b in range(2):
                drain_gather(b)
                start_out(t0 + b, b)
            for b in range(2):
                drain_out(b)
                fire(t0 + 2 + b, b)
            return carry

        lax.fori_loop(0, n_pairs - 1, body, 0)

        t0 = n_chunks - 2
        for b in range(2):
            drain_gather(b)
            start_out(t0 + b, b)
        for b in range(2):
            drain_out(b)

    return emb


def kernel(data, table):
    B0, S = data.shape
    V, D = table.shape
    B = B0 * S
    NW = 32
    assert B % (NW * IDXW) == 0
    rows_per_w = B // (NW * IDXW)
    K = 5
    assert rows_per_w % (2 * K) == 0
    idx2d = data.reshape(B // IDXW, IDXW)
    out = _emb_call(B, D, rows_per_w, K)(table, idx2d)
    return out.reshape(B0, S, D)


# speedup vs baseline: 1.0104x; 1.0007x over previous
import functools

import jax
import jax.numpy as jnp
from jax import lax
from jax.experimental import pallas as pl
from jax.experimental.pallas import tpu as pltpu, tpu_sc as plsc

SPLITS = ((0, 104), (104, 96))


def _emb_call(B0, S, D, NB):
    mesh = plsc.VectorSubcoreMesh(core_axis_name="c", subcore_axis_name="s")
    info = plsc.get_sparse_core_info()
    NC, NS = info.num_cores, info.num_subcores
    NW = NC * NS
    rows_w = B0 // NW
    n_chunks = rows_w // NB

    @functools.partial(
        pl.kernel,
        mesh=mesh,
        out_type=jax.ShapeDtypeStruct((B0, S, D), jnp.float32),
        compiler_params=pltpu.CompilerParams(use_tc_tiling_on_sc=False),
        scratch_types=[
            pltpu.VMEM((NB, S), jnp.int32),
            pltpu.VMEM((NB, S, D), jnp.float32),
            pltpu.SemaphoreType.DMA,
        ],
    )
    def emb(table_hbm, idx_hbm, out_hbm, idx_v, rows_v, sem):
        wid = lax.axis_index("s") * NC + lax.axis_index("c")
        b0w = wid * rows_w

        def body(t, carry):
            b0 = b0w + t * NB
            pltpu.sync_copy(idx_hbm.at[pl.ds(b0, NB)], idx_v)
            cps = []
            for bi in range(NB):
                for off, n in SPLITS:
                    cps.append(
                        pltpu.async_copy(
                            table_hbm.at[idx_v.at[bi, pl.ds(off, n)]],
                            rows_v.at[bi, pl.ds(off, n)],
                            sem,
                        )
                    )
            for cp in cps:
                cp.wait()
            pltpu.sync_copy(rows_v, out_hbm.at[pl.ds(b0, NB)])
            return carry

        lax.fori_loop(0, n_chunks, body, 0)

    return emb


def kernel(data, table):
    B0, S = data.shape
    V, D = table.shape
    NB = 8
    return _emb_call(B0, S, D, NB)(table, data)
